# baseline (speedup 1.0000x reference)
import jax
import jax.numpy as jnp
from jax import lax
from jax.experimental import pallas as pl
from jax.experimental.pallas import tpu as pltpu
from jax.experimental.pallas import tpu_sc as plsc

N = 10000
E = 320000
D_FEAT = 128
D_EDGE = 16
NT_DIM = 64
ET_DIM = 16

NC = 2
NS = 16
NW = NC * NS

E_PER_W = E // NW
ECHUNK = 1024
N_ECHUNKS = -(-E_PER_W // ECHUNK)

NCHUNK = 200
N_NCHUNKS = N // NCHUNK


def _gather_rows(table_hbm, idx_ref, dst, rows, sem):
    copies = []
    off = 0
    while off < rows:
        sub = min(128, rows - off)
        copies.append(
            pltpu.async_copy(
                table_hbm.at[idx_ref.at[pl.ds(off, sub)]],
                dst.at[pl.ds(off, sub)],
                sem,
            )
        )
        off += sub
    return copies


def _sc_body(x_hbm, eattr_hbm, ntypes_hbm, etypes_hbm, ntab_hbm, etab_hbm,
             xcat_hbm, ecat_hbm,
             nidx, nemb, nxf, eidx0, eidx1, eemb0, eemb1, eatt0, eatt1,
             sem_i0, sem_i1, sem_a0, sem_a1, sem_g0, sem_g1,
             sem_s0, sem_s1, sem_n):
    c = lax.axis_index("c")
    s = lax.axis_index("s")
    wid = s * NC + c

    eidx = (eidx0, eidx1)
    eemb = (eemb0, eemb1)
    eatt = (eatt0, eatt1)
    sem_i = (sem_i0, sem_i1)
    sem_a = (sem_a0, sem_a1)
    sem_g = (sem_g0, sem_g1)
    sem_s = (sem_s0, sem_s1)

    for j in range(-(-N_NCHUNKS // NW)):
        k = wid + NW * j

        @pl.when(k < N_NCHUNKS)
        def _():
            nbase = pl.multiple_of(k * NCHUNK, 8)
            ci = pltpu.async_copy(ntypes_hbm.at[pl.ds(nbase, NCHUNK)], nidx,
                                  sem_n)
            cx = pltpu.async_copy(x_hbm.at[pl.ds(nbase, NCHUNK)], nxf, sem_n)
            ci.wait()
            cps = _gather_rows(ntab_hbm, nidx, nemb, NCHUNK, sem_g0)
            for cp in cps:
                cp.wait()
            c1 = pltpu.async_copy(
                nemb, xcat_hbm.at[pl.ds(nbase, NCHUNK), pl.ds(0, NT_DIM)],
                sem_n)
            cx.wait()
            c2 = pltpu.async_copy(
                nxf, xcat_hbm.at[pl.ds(nbase, NCHUNK), pl.ds(NT_DIM, D_FEAT)],
                sem_n)
            c1.wait()
            c2.wait()

    ebase = wid * E_PER_W

    def rows_of(i):
        return min(ECHUNK, E_PER_W - i * ECHUNK)

    def fire_load(i, p):
        base = pl.multiple_of(ebase + i * ECHUNK, 8)
        r = rows_of(i)
        ci = pltpu.async_copy(etypes_hbm.at[pl.ds(base, r)],
                              eidx[p].at[pl.ds(0, r)], sem_i[p])
        ca = pltpu.async_copy(eattr_hbm.at[pl.ds(base, r)],
                              eatt[p].at[pl.ds(0, r)], sem_a[p])
        return ci, ca

    store_cp = [None, None]
    loads = [None, None]
    loads[0] = fire_load(0, 0)

    for i in range(N_ECHUNKS):
        p = i % 2
        r = rows_of(i)
        if i + 1 < N_ECHUNKS:
            if store_cp[1 - p] is not None:
                for cp in store_cp[1 - p]:
                    cp.wait()
                store_cp[1 - p] = None
            loads[1 - p] = fire_load(i + 1, 1 - p)
        ci, ca = loads[p]
        ci.wait()
        cps = _gather_rows(etab_hbm, eidx[p], eemb[p], r, sem_g[p])
        for cp in cps:
            cp.wait()
        ca.wait()
        base = pl.multiple_of(ebase + i * ECHUNK, 8)
        c1 = pltpu.async_copy(
            eemb[p].at[pl.ds(0, r)],
            ecat_hbm.at[pl.ds(base, r), pl.ds(0, ET_DIM)], sem_s[p])
        c2 = pltpu.async_copy(
            eatt[p].at[pl.ds(0, r)],
            ecat_hbm.at[pl.ds(base, r), pl.ds(ET_DIM, D_EDGE)], sem_s[p])
        store_cp[p] = (c1, c2)

    for p in range(2):
        if store_cp[p] is not None:
            for cp in store_cp[p]:
                cp.wait()


@jax.jit
def _run(x, eattr, ntypes, etypes, ntab, etab):
    mesh = plsc.VectorSubcoreMesh(core_axis_name="c", subcore_axis_name="s")
    f = pl.kernel(
        _sc_body,
        out_type=[
            jax.ShapeDtypeStruct((N, NT_DIM + D_FEAT), jnp.float32),
            jax.ShapeDtypeStruct((E, ET_DIM + D_EDGE), jnp.float32),
        ],
        mesh=mesh,
        compiler_params=pltpu.CompilerParams(use_tc_tiling_on_sc=False),
        scratch_types=[
            pltpu.VMEM((NCHUNK,), jnp.int32),
            pltpu.VMEM((NCHUNK, NT_DIM), jnp.float32),
            pltpu.VMEM((NCHUNK, D_FEAT), jnp.float32),
            pltpu.VMEM((ECHUNK,), jnp.int32),
            pltpu.VMEM((ECHUNK,), jnp.int32),
            pltpu.VMEM((ECHUNK, ET_DIM), jnp.float32),
            pltpu.VMEM((ECHUNK, ET_DIM), jnp.float32),
            pltpu.VMEM((ECHUNK, D_EDGE), jnp.float32),
            pltpu.VMEM((ECHUNK, D_EDGE), jnp.float32),
            pltpu.SemaphoreType.DMA,
            pltpu.SemaphoreType.DMA,
            pltpu.SemaphoreType.DMA,
            pltpu.SemaphoreType.DMA,
            pltpu.SemaphoreType.DMA,
            pltpu.SemaphoreType.DMA,
            pltpu.SemaphoreType.DMA,
            pltpu.SemaphoreType.DMA,
            pltpu.SemaphoreType.DMA,
        ],
    )
    return f(x, eattr, ntypes, etypes, ntab, etab)


def kernel(x, eattr, ntypes, etypes, ntype_table, etype_table):
    ntypes = ntypes.astype(jnp.int32)
    etypes = etypes.astype(jnp.int32)
    x_cat, eattr_cat = _run(x, eattr, ntypes, etypes,
                            ntype_table, etype_table)
    return (x_cat, eattr_cat)


# speedup vs baseline: 1.0102x; 1.0102x over previous
import jax
import jax.numpy as jnp
from jax import lax
from jax.experimental import pallas as pl
from jax.experimental.pallas import tpu as pltpu
from jax.experimental.pallas import tpu_sc as plsc

N = 10000
E = 320000
D_FEAT = 128
D_EDGE = 16
NT_DIM = 64
ET_DIM = 16

NC = 2
NS = 16
NW = NC * NS
L = 16

E_PER_W = E // NW
ECHUNK = 512
N_ECHUNKS = -(-E_PER_W // ECHUNK)

NCHUNK = 200
N_NCHUNKS = N // NCHUNK


def _gather_rows(table_hbm, idx_ref, dst, rows, sem):
    copies = []
    off = 0
    while off < rows:
        sub = min(128, rows - off)
        copies.append(
            pltpu.async_copy(
                table_hbm.at[idx_ref.at[pl.ds(off, sub)]],
                dst.at[pl.ds(off, sub)],
                sem,
            )
        )
        off += sub
    return copies


def _sc_body(x_hbm, eattr_hbm, ntypes_hbm, etypes_hbm, ntab_hbm, etab_hbm,
             xcat_hbm, ecat_hbm,
             nidx, nemb, ncat, eidx0, eidx1, eemb0, eemb1, ecat0, ecat1,
             sem_i0, sem_i1, sem_a0, sem_a1, sem_g0, sem_g1,
             sem_s0, sem_s1, sem_n):
    c = lax.axis_index("c")
    s = lax.axis_index("s")
    wid = s * NC + c

    eidx = (eidx0, eidx1)
    eemb = (eemb0, eemb1)
    ecat = (ecat0, ecat1)
    sem_i = (sem_i0, sem_i1)
    sem_a = (sem_a0, sem_a1)
    sem_g = (sem_g0, sem_g1)
    sem_s = (sem_s0, sem_s1)

    for j in range(-(-N_NCHUNKS // NW)):
        k = wid + NW * j

        @pl.when(k < N_NCHUNKS)
        def _():
            nbase = pl.multiple_of(k * NCHUNK, 8)
            ci = pltpu.async_copy(ntypes_hbm.at[pl.ds(nbase, NCHUNK)], nidx,
                                  sem_n)
            cx = pltpu.async_copy(x_hbm.at[pl.ds(nbase, NCHUNK)],
                                  ncat.at[:, pl.ds(NT_DIM, D_FEAT)], sem_n)
            ci.wait()
            cps = _gather_rows(ntab_hbm, nidx, nemb, NCHUNK, sem_g0)
            for cp in cps:
                cp.wait()

            @plsc.parallel_loop(0, NCHUNK, unroll=4)
            def _(i):
                for t in range(NT_DIM // L):
                    ncat[i, pl.ds(t * L, L)] = nemb[i, pl.ds(t * L, L)]

            cx.wait()
            pltpu.async_copy(ncat, xcat_hbm.at[pl.ds(nbase, NCHUNK)],
                             sem_n).wait()

    ebase = wid * E_PER_W

    def rows_of(i):
        return min(ECHUNK, E_PER_W - i * ECHUNK)

    def fire_load(i, p):
        base = pl.multiple_of(ebase + i * ECHUNK, 8)
        r = rows_of(i)
        ci = pltpu.async_copy(etypes_hbm.at[pl.ds(base, r)],
                              eidx[p].at[pl.ds(0, r)], sem_i[p])
        ca = pltpu.async_copy(eattr_hbm.at[pl.ds(base, r)],
                              ecat[p].at[pl.ds(0, r), pl.ds(ET_DIM, D_EDGE)],
                              sem_a[p])
        return ci, ca

    store_cp = [None, None]
    loads = [None, None]
    loads[0] = fire_load(0, 0)

    for i in range(N_ECHUNKS):
        p = i % 2
        r = rows_of(i)
        if i + 1 < N_ECHUNKS:
            if store_cp[1 - p] is not None:
                store_cp[1 - p].wait()
                store_cp[1 - p] = None
            loads[1 - p] = fire_load(i + 1, 1 - p)
        ci, ca = loads[p]
        ci.wait()
        cps = _gather_rows(etab_hbm, eidx[p], eemb[p], r, sem_g[p])
        for cp in cps:
            cp.wait()

        emb_ref = eemb[p]
        cat_ref = ecat[p]

        @plsc.parallel_loop(0, r, unroll=8)
        def _(i2):
            cat_ref[i2, pl.ds(0, ET_DIM)] = emb_ref[i2, :]

        ca.wait()
        base = pl.multiple_of(ebase + i * ECHUNK, 8)
        store_cp[p] = pltpu.async_copy(ecat[p].at[pl.ds(0, r)],
                                       ecat_hbm.at[pl.ds(base, r)], sem_s[p])

    for p in range(2):
        if store_cp[p] is not None:
            store_cp[p].wait()


@jax.jit
def _run(x, eattr, ntypes, etypes, ntab, etab):
    mesh = plsc.VectorSubcoreMesh(core_axis_name="c", subcore_axis_name="s")
    f = pl.kernel(
        _sc_body,
        out_type=[
            jax.ShapeDtypeStruct((N, NT_DIM + D_FEAT), jnp.float32),
            jax.ShapeDtypeStruct((E, ET_DIM + D_EDGE), jnp.float32),
        ],
        mesh=mesh,
        compiler_params=pltpu.CompilerParams(use_tc_tiling_on_sc=False),
        scratch_types=[
            pltpu.VMEM((NCHUNK,), jnp.int32),
            pltpu.VMEM((NCHUNK, NT_DIM), jnp.float32),
            pltpu.VMEM((NCHUNK, NT_DIM + D_FEAT), jnp.float32),
            pltpu.VMEM((ECHUNK,), jnp.int32),
            pltpu.VMEM((ECHUNK,), jnp.int32),
            pltpu.VMEM((ECHUNK, ET_DIM), jnp.float32),
            pltpu.VMEM((ECHUNK, ET_DIM), jnp.float32),
            pltpu.VMEM((ECHUNK, ET_DIM + D_EDGE), jnp.float32),
            pltpu.VMEM((ECHUNK, ET_DIM + D_EDGE), jnp.float32),
            pltpu.SemaphoreType.DMA,
            pltpu.SemaphoreType.DMA,
            pltpu.SemaphoreType.DMA,
            pltpu.SemaphoreType.DMA,
            pltpu.SemaphoreType.DMA,
            pltpu.SemaphoreType.DMA,
            pltpu.SemaphoreType.DMA,
            pltpu.SemaphoreType.DMA,
            pltpu.SemaphoreType.DMA,
        ],
    )
    return f(x, eattr, ntypes, etypes, ntab, etab)


def kernel(x, eattr, ntypes, etypes, ntype_table, etype_table):
    ntypes = ntypes.astype(jnp.int32)
    etypes = etypes.astype(jnp.int32)
    x_cat, eattr_cat = _run(x, eattr, ntypes, etypes,
                            ntype_table, etype_table)
    return (x_cat, eattr_cat)


# speedup vs baseline: 1.8082x; 1.7900x over previous
import jax
import jax.numpy as jnp
from jax import lax
from jax.experimental import pallas as pl
from jax.experimental.pallas import tpu as pltpu
from jax.experimental.pallas import tpu_sc as plsc

N = 10000
E = 320000
D_FEAT = 128
D_EDGE = 16
NT_DIM = 64
ET_DIM = 16

NC = 2
NS = 16
NW = NC * NS
L = 16

NUM_NTYPES = 119
NUM_ETYPES = 22

E_PER_W = E // NW
ECHUNK = 1024
N_ECHUNKS = -(-E_PER_W // ECHUNK)

NCHUNK = 200
N_NCHUNKS = N // NCHUNK


def _emb_blocks(idx_ref, tab_ref, cat_ref, rows, dim, cols):
    iota = lax.iota(jnp.int32, L)
    nfull = rows // L
    rem = rows - nfull * L

    @plsc.parallel_loop(0, nfull, unroll=2)
    def _(b):
        b16 = b * L
        ev = idx_ref[pl.ds(b16, L)]
        rows_v = iota + b16
        for j in range(dim):
            g = plsc.load_gather(tab_ref, [ev, cols[j]])
            plsc.store_scatter(cat_ref, [rows_v, cols[j]], g)

    if rem:
        b16 = rows - L
        ev = idx_ref[pl.ds(b16, L)]
        rows_v = iota + b16
        for j in range(dim):
            g = plsc.load_gather(tab_ref, [ev, cols[j]])
            plsc.store_scatter(cat_ref, [rows_v, cols[j]], g)


def _sc_body(x_hbm, eattr_hbm, ntypes_hbm, etypes_hbm, ntab_hbm, etab_hbm,
             xcat_hbm, ecat_hbm,
             nidx, ncat, ntab_v, etab_v, eidx0, eidx1, ecat0, ecat1,
             sem_i0, sem_i1, sem_a0, sem_a1, sem_t,
             sem_s0, sem_s1, sem_n):
    c = lax.axis_index("c")
    s = lax.axis_index("s")
    wid = s * NC + c

    eidx = (eidx0, eidx1)
    ecat = (ecat0, ecat1)
    sem_i = (sem_i0, sem_i1)
    sem_a = (sem_a0, sem_a1)
    sem_s = (sem_s0, sem_s1)

    ct1 = pltpu.async_copy(ntab_hbm, ntab_v, sem_t)
    ct2 = pltpu.async_copy(etab_hbm, etab_v, sem_t)

    cols = [jnp.full((L,), j, jnp.int32) for j in range(NT_DIM)]

    ct1.wait()
    ct2.wait()

    for j in range(-(-N_NCHUNKS // NW)):
        k = wid + NW * j

        @pl.when(k < N_NCHUNKS)
        def _():
            nbase = pl.multiple_of(k * NCHUNK, 8)
            ci = pltpu.async_copy(ntypes_hbm.at[pl.ds(nbase, NCHUNK)], nidx,
                                  sem_n)
            cx = pltpu.async_copy(x_hbm.at[pl.ds(nbase, NCHUNK)],
                                  ncat.at[:, pl.ds(NT_DIM, D_FEAT)], sem_n)
            ci.wait()
            _emb_blocks(nidx, ntab_v, ncat, NCHUNK, NT_DIM, cols)
            cx.wait()
            pltpu.async_copy(ncat, xcat_hbm.at[pl.ds(nbase, NCHUNK)],
                             sem_n).wait()

    ebase = wid * E_PER_W

    def rows_of(i):
        return min(ECHUNK, E_PER_W - i * ECHUNK)

    def fire_load(i, p):
        base = pl.multiple_of(ebase + i * ECHUNK, 8)
        r = rows_of(i)
        ci = pltpu.async_copy(etypes_hbm.at[pl.ds(base, r)],
                              eidx[p].at[pl.ds(0, r)], sem_i[p])
        ca = pltpu.async_copy(eattr_hbm.at[pl.ds(base, r)],
                              ecat[p].at[pl.ds(0, r), pl.ds(ET_DIM, D_EDGE)],
                              sem_a[p])
        return ci, ca

    store_cp = [None, None]
    loads = [None, None]
    loads[0] = fire_load(0, 0)

    for i in range(N_ECHUNKS):
        p = i % 2
        r = rows_of(i)
        if i + 1 < N_ECHUNKS:
            if store_cp[1 - p] is not None:
                store_cp[1 - p].wait()
                store_cp[1 - p] = None
            loads[1 - p] = fire_load(i + 1, 1 - p)
        ci, ca = loads[p]
        ci.wait()
        _emb_blocks(eidx[p], etab_v, ecat[p], r, ET_DIM, cols)
        ca.wait()
        base = pl.multiple_of(ebase + i * ECHUNK, 8)
        store_cp[p] = pltpu.async_copy(ecat[p].at[pl.ds(0, r)],
                                       ecat_hbm.at[pl.ds(base, r)], sem_s[p])

    for p in range(2):
        if store_cp[p] is not None:
            store_cp[p].wait()


@jax.jit
def _run(x, eattr, ntypes, etypes, ntab, etab):
    mesh = plsc.VectorSubcoreMesh(core_axis_name="c", subcore_axis_name="s")
    f = pl.kernel(
        _sc_body,
        out_type=[
            jax.ShapeDtypeStruct((N, NT_DIM + D_FEAT), jnp.float32),
            jax.ShapeDtypeStruct((E, ET_DIM + D_EDGE), jnp.float32),
        ],
        mesh=mesh,
        compiler_params=pltpu.CompilerParams(use_tc_tiling_on_sc=False,
                                             needs_layout_passes=False),
        scratch_types=[
            pltpu.VMEM((NCHUNK,), jnp.int32),
            pltpu.VMEM((NCHUNK, NT_DIM + D_FEAT), jnp.float32),
            pltpu.VMEM((NUM_NTYPES, NT_DIM), jnp.float32),
            pltpu.VMEM((NUM_ETYPES, ET_DIM), jnp.float32),
            pltpu.VMEM((ECHUNK,), jnp.int32),
            pltpu.VMEM((ECHUNK,), jnp.int32),
            pltpu.VMEM((ECHUNK, ET_DIM + D_EDGE), jnp.float32),
            pltpu.VMEM((ECHUNK, ET_DIM + D_EDGE), jnp.float32),
            pltpu.SemaphoreType.DMA,
            pltpu.SemaphoreType.DMA,
            pltpu.SemaphoreType.DMA,
            pltpu.SemaphoreType.DMA,
            pltpu.SemaphoreType.DMA,
            pltpu.SemaphoreType.DMA,
            pltpu.SemaphoreType.DMA,
            pltpu.SemaphoreType.DMA,
        ],
    )
    return f(x, eattr, ntypes, etypes, ntab, etab)


def kernel(x, eattr, ntypes, etypes, ntype_table, etype_table):
    ntypes = ntypes.astype(jnp.int32)
    etypes = etypes.astype(jnp.int32)
    x_cat, eattr_cat = _run(x, eattr, ntypes, etypes,
                            ntype_table, etype_table)
    return (x_cat, eattr_cat)


# speedup vs baseline: 5.7669x; 3.1892x over previous
import jax
import jax.numpy as jnp
from jax import lax
from jax.experimental import pallas as pl
from jax.experimental.pallas import tpu as pltpu
from jax.experimental.pallas import tpu_sc as plsc

N = 10000
E = 320000
D_FEAT = 128
D_EDGE = 16
NT_DIM = 64
ET_DIM = 16

NC = 2
NS = 16
NW = NC * NS
L = 16

NUM_NTYPES = 119
NUM_ETYPES = 22

ROW_T = 128
COL_T = 8
E_RT = E // ROW_T
ETJ_IN = D_EDGE // COL_T
ETJ_OUT = (ET_DIM + D_EDGE) // COL_T

ECHUNK_T = 4
ECHUNK = ECHUNK_T * ROW_T
N_ECHUNKS = E // ECHUNK
MAX_SLOTS = -(-N_ECHUNKS // NW)

NCHUNK = 200
N_NCHUNKS = N // NCHUNK


def _sc_body(x_hbm, eattr4_hbm, ntypes_hbm, etypes_hbm, ntab_hbm, etab_hbm,
             xcat_hbm, ecat4_hbm,
             nidx, ncat, ntab_v, etab_v, eidx0, eidx1, ebuf0, ebuf1,
             sem_i0, sem_i1, sem_a0, sem_a1, sem_t,
             sem_s0, sem_s1, sem_n):
    c = lax.axis_index("c")
    s = lax.axis_index("s")
    wid = s * NC + c

    eidx = (eidx0, eidx1)
    ebuf = (ebuf0, ebuf1)
    sem_i = (sem_i0, sem_i1)
    sem_a = (sem_a0, sem_a1)
    sem_s = (sem_s0, sem_s1)

    ct1 = pltpu.async_copy(ntab_hbm, ntab_v, sem_t)
    ct2 = pltpu.async_copy(etab_hbm, etab_v, sem_t)

    iota = lax.iota(jnp.int32, L)
    cols = [jnp.full((L,), j, jnp.int32) for j in range(NT_DIM)]

    ct1.wait()
    ct2.wait()

    for j in range(-(-N_NCHUNKS // NW)):
        k = wid + NW * j

        @pl.when(k < N_NCHUNKS)
        def _():
            nbase = pl.multiple_of(k * NCHUNK, 8)
            ci = pltpu.async_copy(ntypes_hbm.at[k], nidx, sem_n)
            cx = pltpu.async_copy(x_hbm.at[pl.ds(nbase, NCHUNK)],
                                  ncat.at[:, pl.ds(NT_DIM, D_FEAT)], sem_n)
            ci.wait()

            nfull = NCHUNK // L

            @plsc.parallel_loop(0, nfull, unroll=2)
            def _(b):
                b16 = pl.multiple_of(b * L, 8)
                ev = nidx[pl.ds(b16, L)]
                rows_v = iota + b16
                for jc in range(NT_DIM):
                    g = plsc.load_gather(ntab_v, [ev, cols[jc]])
                    plsc.store_scatter(ncat, [rows_v, cols[jc]], g)

            if NCHUNK % L:
                b16 = NCHUNK - L
                ev = nidx[pl.ds(b16, L)]
                rows_v = iota + b16
                for jc in range(NT_DIM):
                    g = plsc.load_gather(ntab_v, [ev, cols[jc]])
                    plsc.store_scatter(ncat, [rows_v, cols[jc]], g)

            cx.wait()
            pltpu.async_copy(ncat, xcat_hbm.at[pl.ds(nbase, NCHUNK)],
                             sem_n).wait()

    def fire_load(slot, p):
        k = chunk_of(slot)
        ti0 = k * ECHUNK_T
        ci = pltpu.async_copy(etypes_hbm.at[k], eidx[p], sem_i[p])
        cas = []
        for tj in range(ETJ_IN):
            cas.append(pltpu.async_copy(
                eattr4_hbm.at[tj, pl.ds(ti0, ECHUNK_T)],
                ebuf[p].at[ETJ_IN + tj], sem_a[p]))
        return ci, cas

    def fire_store(slot, p):
        k = chunk_of(slot)
        ti0 = k * ECHUNK_T
        cs = []
        for tj in range(ETJ_OUT):
            cs.append(pltpu.async_copy(
                ebuf[p].at[tj], ecat4_hbm.at[tj, pl.ds(ti0, ECHUNK_T)],
                sem_s[p]))
        return cs

    def chunk_of(slot):
        return jnp.minimum(wid + NW * slot, N_ECHUNKS - 1)

    def process(slot, p):
        ci, cas = loads[p]
        ci.wait()
        buf = ebuf[p]
        idx_ref = eidx[p]

        @plsc.parallel_loop(0, ECHUNK // L, unroll=2)
        def _(t):
            bi = t // (ROW_T // L)
            lg = t % (ROW_T // L)
            ev = idx_ref[pl.ds(pl.multiple_of(t * L, 8), L)]
            for jc in range(ET_DIM):
                g = plsc.load_gather(etab_v, [ev, cols[jc]])
                buf[jc // COL_T, bi, jc % COL_T, pl.ds(lg * L, L)] = g

        for ca in cas:
            ca.wait()
        store_cp[p] = fire_store(slot, p)

    store_cp = [None, None]
    loads = [None, None]
    loads[0] = fire_load(0, 0)

    for slot in range(MAX_SLOTS):
        p = slot % 2
        if slot + 1 < MAX_SLOTS:
            if store_cp[1 - p] is not None:
                for cp in store_cp[1 - p]:
                    cp.wait()
                store_cp[1 - p] = None
            loads[1 - p] = fire_load(slot + 1, 1 - p)
        process(slot, p)

    for p in range(2):
        if store_cp[p] is not None:
            for cp in store_cp[p]:
                cp.wait()


@jax.jit
def _run(x, eattr4, ntypes, etypes, ntab, etab):
    mesh = plsc.VectorSubcoreMesh(core_axis_name="c", subcore_axis_name="s")
    f = pl.kernel(
        _sc_body,
        out_type=[
            jax.ShapeDtypeStruct((N, NT_DIM + D_FEAT), jnp.float32),
            jax.ShapeDtypeStruct((ETJ_OUT, E_RT, COL_T, ROW_T), jnp.float32),
        ],
        mesh=mesh,
        compiler_params=pltpu.CompilerParams(use_tc_tiling_on_sc=False,
                                             needs_layout_passes=False),
        scratch_types=[
            pltpu.VMEM((NCHUNK,), jnp.int32),
            pltpu.VMEM((NCHUNK, NT_DIM + D_FEAT), jnp.float32),
            pltpu.VMEM((NUM_NTYPES, NT_DIM), jnp.float32),
            pltpu.VMEM((NUM_ETYPES, ET_DIM), jnp.float32),
            pltpu.VMEM((ECHUNK,), jnp.int32),
            pltpu.VMEM((ECHUNK,), jnp.int32),
            pltpu.VMEM((ETJ_OUT, ECHUNK_T, COL_T, ROW_T), jnp.float32),
            pltpu.VMEM((ETJ_OUT, ECHUNK_T, COL_T, ROW_T), jnp.float32),
            pltpu.SemaphoreType.DMA,
            pltpu.SemaphoreType.DMA,
            pltpu.SemaphoreType.DMA,
            pltpu.SemaphoreType.DMA,
            pltpu.SemaphoreType.DMA,
            pltpu.SemaphoreType.DMA,
            pltpu.SemaphoreType.DMA,
            pltpu.SemaphoreType.DMA,
        ],
    )
    return f(x, eattr4, ntypes, etypes, ntab, etab)


def kernel(x, eattr, ntypes, etypes, ntype_table, etype_table):
    ntypes = ntypes.astype(jnp.int32).reshape(N_NCHUNKS, NCHUNK)
    etypes = etypes.astype(jnp.int32).reshape(N_ECHUNKS, ECHUNK)
    eattr4 = eattr.reshape(E_RT, ROW_T, ETJ_IN, COL_T).transpose(2, 0, 3, 1)
    x_cat, ecat4 = _run(x, eattr4, ntypes, etypes,
                        ntype_table, etype_table)
    eattr_cat = ecat4.transpose(1, 3, 0, 2).reshape(E, ET_DIM + D_EDGE)
    return (x_cat, eattr_cat)


# speedup vs baseline: 7.8387x; 1.3593x over previous
import jax
import jax.numpy as jnp
from jax import lax
from jax.experimental import pallas as pl
from jax.experimental.pallas import tpu as pltpu
from jax.experimental.pallas import tpu_sc as plsc

N = 10000
E = 320000
D_FEAT = 128
D_EDGE = 16
NT_DIM = 64
ET_DIM = 16

NC = 2
NS = 16
NW = NC * NS
L = 16

NUM_NTYPES = 119
NUM_ETYPES = 22

ROW_T = 128
COL_T = 8
E_RT = E // ROW_T
ETJ_IN = D_EDGE // COL_T
ETJ_OUT = (ET_DIM + D_EDGE) // COL_T

ECHUNK_T = 4
ECHUNK = ECHUNK_T * ROW_T
N_ECHUNKS = E // ECHUNK
MAX_SLOTS = -(-N_ECHUNKS // NW)

N_RT = -(-N // ROW_T)
NPAD = N_RT * ROW_T
NTJ_OUT = (NT_DIM + D_FEAT) // COL_T
N_NSLOTS = -(-N_RT // NW)


def _sc_body(x_hbm, eattr4_hbm, ntypes_hbm, etypes_hbm, ntab_hbm, etab_hbm,
             xcat4_hbm, ecat4_hbm,
             nidx, xblk, nbuf, ntab_v, etab_v, eidx0, eidx1, ebuf0, ebuf1,
             sem_i0, sem_i1, sem_a0, sem_a1, sem_t,
             sem_s0, sem_s1, sem_n):
    c = lax.axis_index("c")
    s = lax.axis_index("s")
    wid = s * NC + c

    eidx = (eidx0, eidx1)
    ebuf = (ebuf0, ebuf1)
    sem_i = (sem_i0, sem_i1)
    sem_a = (sem_a0, sem_a1)
    sem_s = (sem_s0, sem_s1)

    ct1 = pltpu.async_copy(ntab_hbm, ntab_v, sem_t)
    ct2 = pltpu.async_copy(etab_hbm, etab_v, sem_t)

    iota = lax.iota(jnp.int32, L)
    cols = [jnp.full((L,), j, jnp.int32) for j in range(D_FEAT)]

    ct1.wait()
    ct2.wait()

    for j in range(N_NSLOTS):
        k = jnp.minimum(wid + NW * j, N_RT - 1)
        ci = pltpu.async_copy(ntypes_hbm.at[k], nidx, sem_n)
        cx = pltpu.async_copy(
            x_hbm.at[pl.ds(pl.multiple_of(k * ROW_T, ROW_T), ROW_T)],
            xblk, sem_n)
        ci.wait()

        @plsc.parallel_loop(0, ROW_T // L, unroll=1)
        def _(lg):
            ev = nidx[pl.ds(pl.multiple_of(lg * L, 8), L)]
            for jc in range(NT_DIM):
                g = plsc.load_gather(ntab_v, [ev, cols[jc]])
                nbuf[jc // COL_T, jc % COL_T, pl.ds(lg * L, L)] = g

        cx.wait()

        @plsc.parallel_loop(0, D_FEAT, unroll=2)
        def _(f):
            col_v = jnp.full((L,), 0, jnp.int32) + f
            tj = NT_DIM // COL_T + f // COL_T
            sj = f % COL_T
            for lg in range(ROW_T // L):
                g = plsc.load_gather(xblk, [iota + lg * L, col_v])
                nbuf[tj, sj, pl.ds(lg * L, L)] = g

        css = [pltpu.async_copy(nbuf.at[tj], xcat4_hbm.at[tj, k], sem_n)
               for tj in range(NTJ_OUT)]
        for cs in css:
            cs.wait()

    def fire_load(slot, p):
        k = chunk_of(slot)
        ti0 = k * ECHUNK_T
        ci = pltpu.async_copy(etypes_hbm.at[k], eidx[p], sem_i[p])
        cas = []
        for tj in range(ETJ_IN):
            cas.append(pltpu.async_copy(
                eattr4_hbm.at[tj, pl.ds(ti0, ECHUNK_T)],
                ebuf[p].at[ETJ_IN + tj], sem_a[p]))
        return ci, cas

    def fire_store(slot, p):
        k = chunk_of(slot)
        ti0 = k * ECHUNK_T
        cs = []
        for tj in range(ETJ_OUT):
            cs.append(pltpu.async_copy(
                ebuf[p].at[tj], ecat4_hbm.at[tj, pl.ds(ti0, ECHUNK_T)],
                sem_s[p]))
        return cs

    def chunk_of(slot):
        return jnp.minimum(wid + NW * slot, N_ECHUNKS - 1)

    def process(slot, p):
        ci, cas = loads[p]
        ci.wait()
        buf = ebuf[p]
        idx_ref = eidx[p]

        @plsc.parallel_loop(0, ECHUNK // L, unroll=2)
        def _(t):
            bi = t // (ROW_T // L)
            lg = t % (ROW_T // L)
            ev = idx_ref[pl.ds(pl.multiple_of(t * L, 8), L)]
            for jc in range(ET_DIM):
                g = plsc.load_gather(etab_v, [ev, cols[jc]])
                buf[jc // COL_T, bi, jc % COL_T, pl.ds(lg * L, L)] = g

        for ca in cas:
            ca.wait()
        store_cp[p] = fire_store(slot, p)

    store_cp = [None, None]
    loads = [None, None]
    loads[0] = fire_load(0, 0)

    for slot in range(MAX_SLOTS):
        p = slot % 2
        if slot + 1 < MAX_SLOTS:
            if store_cp[1 - p] is not None:
                for cp in store_cp[1 - p]:
                    cp.wait()
                store_cp[1 - p] = None
            loads[1 - p] = fire_load(slot + 1, 1 - p)
        process(slot, p)

    for p in range(2):
        if store_cp[p] is not None:
            for cp in store_cp[p]:
                cp.wait()


@jax.jit
def _run(x, eattr4, ntypes, etypes, ntab, etab):
    mesh = plsc.VectorSubcoreMesh(core_axis_name="c", subcore_axis_name="s")
    f = pl.kernel(
        _sc_body,
        out_type=[
            jax.ShapeDtypeStruct((NTJ_OUT, N_RT, COL_T, ROW_T), jnp.float32),
            jax.ShapeDtypeStruct((ETJ_OUT, E_RT, COL_T, ROW_T), jnp.float32),
        ],
        mesh=mesh,
        compiler_params=pltpu.CompilerParams(use_tc_tiling_on_sc=False,
                                             needs_layout_passes=False),
        scratch_types=[
            pltpu.VMEM((ROW_T,), jnp.int32),
            pltpu.VMEM((ROW_T, D_FEAT), jnp.float32),
            pltpu.VMEM((NTJ_OUT, COL_T, ROW_T), jnp.float32),
            pltpu.VMEM((NUM_NTYPES, NT_DIM), jnp.float32),
            pltpu.VMEM((NUM_ETYPES, ET_DIM), jnp.float32),
            pltpu.VMEM((ECHUNK,), jnp.int32),
            pltpu.VMEM((ECHUNK,), jnp.int32),
            pltpu.VMEM((ETJ_OUT, ECHUNK_T, COL_T, ROW_T), jnp.float32),
            pltpu.VMEM((ETJ_OUT, ECHUNK_T, COL_T, ROW_T), jnp.float32),
            pltpu.SemaphoreType.DMA,
            pltpu.SemaphoreType.DMA,
            pltpu.SemaphoreType.DMA,
            pltpu.SemaphoreType.DMA,
            pltpu.SemaphoreType.DMA,
            pltpu.SemaphoreType.DMA,
            pltpu.SemaphoreType.DMA,
            pltpu.SemaphoreType.DMA,
        ],
    )
    return f(x, eattr4, ntypes, etypes, ntab, etab)


def kernel(x, eattr, ntypes, etypes, ntype_table, etype_table):
    ntypes = jnp.pad(ntypes.astype(jnp.int32),
                     (0, NPAD - N)).reshape(N_RT, ROW_T)
    etypes = etypes.astype(jnp.int32).reshape(N_ECHUNKS, ECHUNK)
    x_pad = jnp.pad(x, ((0, NPAD - N), (0, 0)))
    eattr4 = eattr.reshape(E_RT, ROW_T, ETJ_IN, COL_T).transpose(2, 0, 3, 1)
    xcat4, ecat4 = _run(x_pad, eattr4, ntypes, etypes,
                        ntype_table, etype_table)
    x_cat = xcat4.transpose(1, 3, 0, 2).reshape(NPAD, NT_DIM + D_FEAT)[:N]
    eattr_cat = ecat4.transpose(1, 3, 0, 2).reshape(E, ET_DIM + D_EDGE)
    return (x_cat, eattr_cat)


# speedup vs baseline: 8.1770x; 1.0432x over previous
import jax
import jax.numpy as jnp
from jax import lax
from jax.experimental import pallas as pl
from jax.experimental.pallas import tpu as pltpu
from jax.experimental.pallas import tpu_sc as plsc

N = 10000
E = 320000
D_FEAT = 128
D_EDGE = 16
NT_DIM = 64
ET_DIM = 16

NC = 2
NS = 16
NW = NC * NS
L = 16

NUM_NTYPES = 119
NUM_ETYPES = 22

ROW_T = 128
COL_T = 8
E_RT = E // ROW_T
ETJ_IN = D_EDGE // COL_T
ETJ_OUT = (ET_DIM + D_EDGE) // COL_T

ECHUNK_T = 4
ECHUNK = ECHUNK_T * ROW_T
N_ECHUNKS = E // ECHUNK
MAX_SLOTS = -(-N_ECHUNKS // NW)

N_RT = -(-N // ROW_T)
NPAD = N_RT * ROW_T
NTJ_OUT = (NT_DIM + D_FEAT) // COL_T
N_NSLOTS = -(-N_RT // NW)


def _sc_body(x_hbm, eattr4_hbm, ntypes_hbm, etypes_hbm, ntab_hbm, etab_hbm,
             xcat4_hbm, ecat4_hbm,
             nidx, xblk, nbuf, ntab_v, etab_v, eidx0, eidx1, ebuf0, ebuf1,
             sem_i0, sem_i1, sem_a0, sem_a1, sem_t,
             sem_s0, sem_s1, sem_n, sem_ni, sem_ns):
    c = lax.axis_index("c")
    s = lax.axis_index("s")
    wid = s * NC + c

    eidx = (eidx0, eidx1)
    ebuf = (ebuf0, ebuf1)
    sem_i = (sem_i0, sem_i1)
    sem_a = (sem_a0, sem_a1)
    sem_s = (sem_s0, sem_s1)

    ct1 = pltpu.async_copy(ntab_hbm, ntab_v, sem_t)
    ct2 = pltpu.async_copy(etab_hbm, etab_v, sem_t)

    iota = lax.iota(jnp.int32, L)
    cols = [jnp.full((L,), j, jnp.int32) for j in range(D_FEAT)]

    def fire_node_loads(j):
        k = jnp.minimum(wid + NW * j, N_RT - 1)
        ci = pltpu.async_copy(ntypes_hbm.at[k], nidx, sem_ni)
        cx = pltpu.async_copy(
            x_hbm.at[pl.ds(pl.multiple_of(k * ROW_T, ROW_T), ROW_T)],
            xblk, sem_n)
        return ci, cx, k

    node_state = {"loads": None, "store": None}

    def do_node(j):
        ci, cx, k = node_state["loads"]
        if node_state["store"] is not None:
            for cs in node_state["store"]:
                cs.wait()
        ci.wait()

        @plsc.parallel_loop(0, ROW_T // L, unroll=1)
        def _(lg):
            ev = nidx[pl.ds(pl.multiple_of(lg * L, 8), L)]
            for jc in range(NT_DIM):
                g = plsc.load_gather(ntab_v, [ev, cols[jc]])
                nbuf[jc // COL_T, jc % COL_T, pl.ds(lg * L, L)] = g

        cx.wait()

        @plsc.parallel_loop(0, D_FEAT, unroll=2)
        def _(f):
            col_v = jnp.full((L,), 0, jnp.int32) + f
            tj = NT_DIM // COL_T + f // COL_T
            sj = f % COL_T
            for lg in range(ROW_T // L):
                g = plsc.load_gather(xblk, [iota + lg * L, col_v])
                nbuf[tj, sj, pl.ds(lg * L, L)] = g

        node_state["store"] = [
            pltpu.async_copy(nbuf.at[tj], xcat4_hbm.at[tj, k], sem_ns)
            for tj in range(NTJ_OUT)]
        if j + 1 < N_NSLOTS:
            node_state["loads"] = fire_node_loads(j + 1)

    def fire_load(slot, p):
        k = chunk_of(slot)
        ti0 = k * ECHUNK_T
        ci = pltpu.async_copy(etypes_hbm.at[k], eidx[p], sem_i[p])
        cas = []
        for tj in range(ETJ_IN):
            cas.append(pltpu.async_copy(
                eattr4_hbm.at[tj, pl.ds(ti0, ECHUNK_T)],
                ebuf[p].at[ETJ_IN + tj], sem_a[p]))
        return ci, cas

    def fire_store(slot, p):
        k = chunk_of(slot)
        ti0 = k * ECHUNK_T
        cs = []
        for tj in range(ETJ_OUT):
            cs.append(pltpu.async_copy(
                ebuf[p].at[tj], ecat4_hbm.at[tj, pl.ds(ti0, ECHUNK_T)],
                sem_s[p]))
        return cs

    def chunk_of(slot):
        return jnp.minimum(wid + NW * slot, N_ECHUNKS - 1)

    def process(slot, p):
        ci, cas = loads[p]
        ci.wait()
        buf = ebuf[p]
        idx_ref = eidx[p]

        @plsc.parallel_loop(0, ECHUNK // L, unroll=2)
        def _(t):
            bi = t // (ROW_T // L)
            lg = t % (ROW_T // L)
            ev = idx_ref[pl.ds(pl.multiple_of(t * L, 8), L)]
            for jc in range(ET_DIM):
                g = plsc.load_gather(etab_v, [ev, cols[jc]])
                buf[jc // COL_T, bi, jc % COL_T, pl.ds(lg * L, L)] = g

        for ca in cas:
            ca.wait()
        store_cp[p] = fire_store(slot, p)

    store_cp = [None, None]
    loads = [None, None]
    loads[0] = fire_load(0, 0)
    node_state["loads"] = fire_node_loads(0)

    ct1.wait()
    ct2.wait()

    for slot in range(MAX_SLOTS):
        p = slot % 2
        if slot + 1 < MAX_SLOTS:
            if store_cp[1 - p] is not None:
                for cp in store_cp[1 - p]:
                    cp.wait()
                store_cp[1 - p] = None
            loads[1 - p] = fire_load(slot + 1, 1 - p)
        if slot < N_NSLOTS:
            do_node(slot)
        process(slot, p)

    for cs in node_state["store"]:
        cs.wait()
    for p in range(2):
        if store_cp[p] is not None:
            for cp in store_cp[p]:
                cp.wait()


@jax.jit
def _run(x, eattr4, ntypes, etypes, ntab, etab):
    mesh = plsc.VectorSubcoreMesh(core_axis_name="c", subcore_axis_name="s")
    f = pl.kernel(
        _sc_body,
        out_type=[
            jax.ShapeDtypeStruct((NTJ_OUT, N_RT, COL_T, ROW_T), jnp.float32),
            jax.ShapeDtypeStruct((ETJ_OUT, E_RT, COL_T, ROW_T), jnp.float32),
        ],
        mesh=mesh,
        compiler_params=pltpu.CompilerParams(use_tc_tiling_on_sc=False,
                                             needs_layout_passes=False),
        scratch_types=[
            pltpu.VMEM((ROW_T,), jnp.int32),
            pltpu.VMEM((ROW_T, D_FEAT), jnp.float32),
            pltpu.VMEM((NTJ_OUT, COL_T, ROW_T), jnp.float32),
            pltpu.VMEM((NUM_NTYPES, NT_DIM), jnp.float32),
            pltpu.VMEM((NUM_ETYPES, ET_DIM), jnp.float32),
            pltpu.VMEM((ECHUNK,), jnp.int32),
            pltpu.VMEM((ECHUNK,), jnp.int32),
            pltpu.VMEM((ETJ_OUT, ECHUNK_T, COL_T, ROW_T), jnp.float32),
            pltpu.VMEM((ETJ_OUT, ECHUNK_T, COL_T, ROW_T), jnp.float32),
            pltpu.SemaphoreType.DMA,
            pltpu.SemaphoreType.DMA,
            pltpu.SemaphoreType.DMA,
            pltpu.SemaphoreType.DMA,
            pltpu.SemaphoreType.DMA,
            pltpu.SemaphoreType.DMA,
            pltpu.SemaphoreType.DMA,
            pltpu.SemaphoreType.DMA,
            pltpu.SemaphoreType.DMA,
            pltpu.SemaphoreType.DMA,
        ],
    )
    return f(x, eattr4, ntypes, etypes, ntab, etab)


def kernel(x, eattr, ntypes, etypes, ntype_table, etype_table):
    ntypes = jnp.pad(ntypes.astype(jnp.int32),
                     (0, NPAD - N)).reshape(N_RT, ROW_T)
    etypes = etypes.astype(jnp.int32).reshape(N_ECHUNKS, ECHUNK)
    x_pad = jnp.pad(x, ((0, NPAD - N), (0, 0)))
    eattr4 = eattr.reshape(E_RT, ROW_T, ETJ_IN, COL_T).transpose(2, 0, 3, 1)
    xcat4, ecat4 = _run(x_pad, eattr4, ntypes, etypes,
                        ntype_table, etype_table)
    x_cat = xcat4.transpose(1, 3, 0, 2).reshape(NPAD, NT_DIM + D_FEAT)[:N]
    eattr_cat = ecat4.transpose(1, 3, 0, 2).reshape(E, ET_DIM + D_EDGE)
    return (x_cat, eattr_cat)


# speedup vs baseline: 8.2905x; 1.0139x over previous
import jax
import jax.numpy as jnp
from jax import lax
from jax.experimental import pallas as pl
from jax.experimental.pallas import tpu as pltpu
from jax.experimental.pallas import tpu_sc as plsc

N = 10000
E = 320000
D_FEAT = 128
D_EDGE = 16
NT_DIM = 64
ET_DIM = 16

NC = 2
NS = 16
NW = NC * NS
L = 16

NUM_NTYPES = 119
NUM_ETYPES = 22

ROW_T = 128
COL_T = 8
E_RT = E // ROW_T
ETJ_IN = D_EDGE // COL_T
ETJ_OUT = (ET_DIM + D_EDGE) // COL_T

ECHUNK_T = 4
ECHUNK = ECHUNK_T * ROW_T
N_ECHUNKS = E // ECHUNK
MAX_SLOTS = -(-N_ECHUNKS // NW)

N_RT = -(-N // ROW_T)
NPAD = N_RT * ROW_T
NTJ_OUT = (NT_DIM + D_FEAT) // COL_T
N_NSLOTS = -(-N_RT // NW)


def _sc_body(x_hbm, eattr4_hbm, ntypes_hbm, etypes_hbm, ntab_hbm, etab_hbm,
             xcat4_hbm, ecat4_hbm,
             nidx, xblk, nbuf, ntab_v, etab_v, eidx0, eidx1, ebuf0, ebuf1,
             sem_i0, sem_i1, sem_a0, sem_a1, sem_t,
             sem_s0, sem_s1, sem_n, sem_ni, sem_ns):
    c = lax.axis_index("c")
    s = lax.axis_index("s")
    wid = s * NC + c

    eidx = (eidx0, eidx1)
    ebuf = (ebuf0, ebuf1)
    sem_i = (sem_i0, sem_i1)
    sem_a = (sem_a0, sem_a1)
    sem_s = (sem_s0, sem_s1)

    ct1 = pltpu.async_copy(ntab_hbm, ntab_v, sem_t)
    ct2 = pltpu.async_copy(etab_hbm, etab_v, sem_t)

    iota = lax.iota(jnp.int32, L)
    cols = [jnp.full((L,), j, jnp.int32) for j in range(D_FEAT)]

    def fire_node_loads(j):
        k = jnp.minimum(wid + NW * j, N_RT - 1)
        ci = pltpu.async_copy(ntypes_hbm.at[k], nidx, sem_ni)
        cx = pltpu.async_copy(
            x_hbm.at[pl.ds(pl.multiple_of(k * ROW_T, ROW_T), ROW_T)],
            xblk, sem_n)
        return ci, cx, k

    node_state = {"loads": None, "store": None}

    def do_node(j):
        ci, cx, k = node_state["loads"]
        if node_state["store"] is not None:
            for cs in node_state["store"]:
                cs.wait()
        ci.wait()

        @plsc.parallel_loop(0, ROW_T // L, unroll=1)
        def _(lg):
            ev = nidx[pl.ds(pl.multiple_of(lg * L, 8), L)]
            for jc in range(NT_DIM):
                g = plsc.load_gather(ntab_v, [ev, cols[jc]])
                nbuf[jc // COL_T, jc % COL_T, pl.ds(lg * L, L)] = g

        cx.wait()

        @plsc.parallel_loop(0, D_FEAT, unroll=2)
        def _(f):
            col_v = jnp.full((L,), 0, jnp.int32) + f
            tj = NT_DIM // COL_T + f // COL_T
            sj = f % COL_T
            for lg in range(ROW_T // L):
                g = plsc.load_gather(xblk, [iota + lg * L, col_v])
                nbuf[tj, sj, pl.ds(lg * L, L)] = g

        node_state["store"] = [
            pltpu.async_copy(nbuf, xcat4_hbm.at[:, k], sem_ns)]
        if j + 1 < N_NSLOTS:
            node_state["loads"] = fire_node_loads(j + 1)

    def fire_load(slot, p):
        k = chunk_of(slot)
        ti0 = k * ECHUNK_T
        ci = pltpu.async_copy(etypes_hbm.at[k], eidx[p], sem_i[p])
        cas = [pltpu.async_copy(
            eattr4_hbm.at[:, pl.ds(ti0, ECHUNK_T)],
            ebuf[p].at[pl.ds(ETJ_IN, ETJ_IN)], sem_a[p])]
        return ci, cas

    def fire_store(slot, p):
        k = chunk_of(slot)
        ti0 = k * ECHUNK_T
        return [pltpu.async_copy(
            ebuf[p], ecat4_hbm.at[:, pl.ds(ti0, ECHUNK_T)], sem_s[p])]

    def chunk_of(slot):
        return jnp.minimum(wid + NW * slot, N_ECHUNKS - 1)

    def process(slot, p):
        ci, cas = loads[p]
        ci.wait()
        buf = ebuf[p]
        idx_ref = eidx[p]

        @plsc.parallel_loop(0, ECHUNK // L, unroll=2)
        def _(t):
            bi = t // (ROW_T // L)
            lg = t % (ROW_T // L)
            ev = idx_ref[pl.ds(pl.multiple_of(t * L, 8), L)]
            for jc in range(ET_DIM):
                g = plsc.load_gather(etab_v, [ev, cols[jc]])
                buf[jc // COL_T, bi, jc % COL_T, pl.ds(lg * L, L)] = g

        for ca in cas:
            ca.wait()
        store_cp[p] = fire_store(slot, p)

    store_cp = [None, None]
    loads = [None, None]
    loads[0] = fire_load(0, 0)
    node_state["loads"] = fire_node_loads(0)

    ct1.wait()
    ct2.wait()

    for slot in range(MAX_SLOTS):
        p = slot % 2
        if slot + 1 < MAX_SLOTS:
            if store_cp[1 - p] is not None:
                for cp in store_cp[1 - p]:
                    cp.wait()
                store_cp[1 - p] = None
            loads[1 - p] = fire_load(slot + 1, 1 - p)
        if slot < N_NSLOTS:
            do_node(slot)
        process(slot, p)

    for cs in node_state["store"]:
        cs.wait()
    for p in range(2):
        if store_cp[p] is not None:
            for cp in store_cp[p]:
                cp.wait()


@jax.jit
def _run(x, eattr4, ntypes, etypes, ntab, etab):
    mesh = plsc.VectorSubcoreMesh(core_axis_name="c", subcore_axis_name="s")
    f = pl.kernel(
        _sc_body,
        out_type=[
            jax.ShapeDtypeStruct((NTJ_OUT, N_RT, COL_T, ROW_T), jnp.float32),
            jax.ShapeDtypeStruct((ETJ_OUT, E_RT, COL_T, ROW_T), jnp.float32),
        ],
        mesh=mesh,
        compiler_params=pltpu.CompilerParams(use_tc_tiling_on_sc=False,
                                             needs_layout_passes=False),
        scratch_types=[
            pltpu.VMEM((ROW_T,), jnp.int32),
            pltpu.VMEM((ROW_T, D_FEAT), jnp.float32),
            pltpu.VMEM((NTJ_OUT, COL_T, ROW_T), jnp.float32),
            pltpu.VMEM((NUM_NTYPES, NT_DIM), jnp.float32),
            pltpu.VMEM((NUM_ETYPES, ET_DIM), jnp.float32),
            pltpu.VMEM((ECHUNK,), jnp.int32),
            pltpu.VMEM((ECHUNK,), jnp.int32),
            pltpu.VMEM((ETJ_OUT, ECHUNK_T, COL_T, ROW_T), jnp.float32),
            pltpu.VMEM((ETJ_OUT, ECHUNK_T, COL_T, ROW_T), jnp.float32),
            pltpu.SemaphoreType.DMA,
            pltpu.SemaphoreType.DMA,
            pltpu.SemaphoreType.DMA,
            pltpu.SemaphoreType.DMA,
            pltpu.SemaphoreType.DMA,
            pltpu.SemaphoreType.DMA,
            pltpu.SemaphoreType.DMA,
            pltpu.SemaphoreType.DMA,
            pltpu.SemaphoreType.DMA,
            pltpu.SemaphoreType.DMA,
        ],
    )
    return f(x, eattr4, ntypes, etypes, ntab, etab)


def kernel(x, eattr, ntypes, etypes, ntype_table, etype_table):
    ntypes = jnp.pad(ntypes.astype(jnp.int32),
                     (0, NPAD - N)).reshape(N_RT, ROW_T)
    etypes = etypes.astype(jnp.int32).reshape(N_ECHUNKS, ECHUNK)
    x_pad = jnp.pad(x, ((0, NPAD - N), (0, 0)))
    eattr4 = eattr.reshape(E_RT, ROW_T, ETJ_IN, COL_T).transpose(2, 0, 3, 1)
    xcat4, ecat4 = _run(x_pad, eattr4, ntypes, etypes,
                        ntype_table, etype_table)
    x_cat = xcat4.transpose(1, 3, 0, 2).reshape(NPAD, NT_DIM + D_FEAT)[:N]
    eattr_cat = ecat4.transpose(1, 3, 0, 2).reshape(E, ET_DIM + D_EDGE)
    return (x_cat, eattr_cat)


# speedup vs baseline: 8.4489x; 1.0191x over previous
import jax
import jax.numpy as jnp
from jax import lax
from jax.experimental import pallas as pl
from jax.experimental.pallas import tpu as pltpu
from jax.experimental.pallas import tpu_sc as plsc

N = 10000
E = 320000
D_FEAT = 128
D_EDGE = 16
NT_DIM = 64
ET_DIM = 16

NC = 2
NS = 16
NW = NC * NS
L = 16

NUM_NTYPES = 119
NUM_ETYPES = 22

ROW_T = 128
COL_T = 8
E_RT = E // ROW_T
ETJ_IN = D_EDGE // COL_T
ETJ_OUT = (ET_DIM + D_EDGE) // COL_T

ECHUNK_T = 5
ECHUNK = ECHUNK_T * ROW_T
N_ECHUNKS = E // ECHUNK
MAX_SLOTS = -(-N_ECHUNKS // NW)

N_RT = -(-N // ROW_T)
NPAD = N_RT * ROW_T
NTJ_OUT = (NT_DIM + D_FEAT) // COL_T
N_NSLOTS = -(-N_RT // NW)


def _sc_body(x_hbm, eattr4_hbm, ntypes_hbm, etypes_hbm, ntab_hbm, etab_hbm,
             xcat4_hbm, ecat4_hbm,
             nidx, xblk, nbuf, ntab_v, etab_v, eidx0, eidx1, ebuf0, ebuf1,
             sem_i0, sem_i1, sem_a0, sem_a1, sem_t,
             sem_s0, sem_s1, sem_n, sem_ni, sem_ns):
    c = lax.axis_index("c")
    s = lax.axis_index("s")
    wid = s * NC + c

    eidx = (eidx0, eidx1)
    ebuf = (ebuf0, ebuf1)
    sem_i = (sem_i0, sem_i1)
    sem_a = (sem_a0, sem_a1)
    sem_s = (sem_s0, sem_s1)

    ct1 = pltpu.async_copy(ntab_hbm, ntab_v, sem_t)
    ct2 = pltpu.async_copy(etab_hbm, etab_v, sem_t)

    iota = lax.iota(jnp.int32, L)
    cols = [jnp.full((L,), j, jnp.int32) for j in range(D_FEAT)]

    def fire_node_loads(j):
        k = jnp.minimum(wid + NW * j, N_RT - 1)
        ci = pltpu.async_copy(ntypes_hbm.at[k], nidx, sem_ni)
        cx = pltpu.async_copy(
            x_hbm.at[pl.ds(pl.multiple_of(k * ROW_T, ROW_T), ROW_T)],
            xblk, sem_n)
        return ci, cx, k

    node_state = {"loads": None, "store": None}

    def do_node(j):
        ci, cx, k = node_state["loads"]
        if node_state["store"] is not None:
            for cs in node_state["store"]:
                cs.wait()
        ci.wait()

        @plsc.parallel_loop(0, ROW_T // L, unroll=1)
        def _(lg):
            ev = nidx[pl.ds(pl.multiple_of(lg * L, 8), L)]
            for jc in range(NT_DIM):
                g = plsc.load_gather(ntab_v, [ev, cols[jc]])
                nbuf[jc // COL_T, jc % COL_T, pl.ds(lg * L, L)] = g

        cx.wait()

        @plsc.parallel_loop(0, D_FEAT, unroll=2)
        def _(f):
            col_v = jnp.full((L,), 0, jnp.int32) + f
            tj = NT_DIM // COL_T + f // COL_T
            sj = f % COL_T
            for lg in range(ROW_T // L):
                g = plsc.load_gather(xblk, [iota + lg * L, col_v])
                nbuf[tj, sj, pl.ds(lg * L, L)] = g

        node_state["store"] = [
            pltpu.async_copy(nbuf, xcat4_hbm.at[:, k], sem_ns)]
        if j + 1 < N_NSLOTS:
            node_state["loads"] = fire_node_loads(j + 1)

    def fire_load(slot, p):
        k = chunk_of(slot)
        ti0 = k * ECHUNK_T
        ci = pltpu.async_copy(etypes_hbm.at[k], eidx[p], sem_i[p])
        cas = [pltpu.async_copy(
            eattr4_hbm.at[:, pl.ds(ti0, ECHUNK_T)],
            ebuf[p].at[pl.ds(ETJ_IN, ETJ_IN)], sem_a[p])]
        return ci, cas

    def fire_store(slot, p):
        k = chunk_of(slot)
        ti0 = k * ECHUNK_T
        return [pltpu.async_copy(
            ebuf[p], ecat4_hbm.at[:, pl.ds(ti0, ECHUNK_T)], sem_s[p])]

    def chunk_of(slot):
        return jnp.minimum(wid + NW * slot, N_ECHUNKS - 1)

    def process(slot, p):
        ci, cas = loads[p]
        ci.wait()
        buf = ebuf[p]
        idx_ref = eidx[p]

        @plsc.parallel_loop(0, ECHUNK // L, unroll=2)
        def _(t):
            bi = t // (ROW_T // L)
            lg = t % (ROW_T // L)
            ev = idx_ref[pl.ds(pl.multiple_of(t * L, 8), L)]
            for jc in range(ET_DIM):
                g = plsc.load_gather(etab_v, [ev, cols[jc]])
                buf[jc // COL_T, bi, jc % COL_T, pl.ds(lg * L, L)] = g

        for ca in cas:
            ca.wait()
        store_cp[p] = fire_store(slot, p)

    store_cp = [None, None]
    loads = [None, None]
    loads[0] = fire_load(0, 0)
    node_state["loads"] = fire_node_loads(0)

    ct1.wait()
    ct2.wait()

    for slot in range(MAX_SLOTS):
        p = slot % 2
        if slot + 1 < MAX_SLOTS:
            if store_cp[1 - p] is not None:
                for cp in store_cp[1 - p]:
                    cp.wait()
                store_cp[1 - p] = None
            loads[1 - p] = fire_load(slot + 1, 1 - p)
        if slot < N_NSLOTS:
            do_node(slot)
        process(slot, p)

    for cs in node_state["store"]:
        cs.wait()
    for p in range(2):
        if store_cp[p] is not None:
            for cp in store_cp[p]:
                cp.wait()


@jax.jit
def _run(x, eattr4, ntypes, etypes, ntab, etab):
    mesh = plsc.VectorSubcoreMesh(core_axis_name="c", subcore_axis_name="s")
    f = pl.kernel(
        _sc_body,
        out_type=[
            jax.ShapeDtypeStruct((NTJ_OUT, N_RT, COL_T, ROW_T), jnp.float32),
            jax.ShapeDtypeStruct((ETJ_OUT, E_RT, COL_T, ROW_T), jnp.float32),
        ],
        mesh=mesh,
        compiler_params=pltpu.CompilerParams(use_tc_tiling_on_sc=False,
                                             needs_layout_passes=False),
        scratch_types=[
            pltpu.VMEM((ROW_T,), jnp.int32),
            pltpu.VMEM((ROW_T, D_FEAT), jnp.float32),
            pltpu.VMEM((NTJ_OUT, COL_T, ROW_T), jnp.float32),
            pltpu.VMEM((NUM_NTYPES, NT_DIM), jnp.float32),
            pltpu.VMEM((NUM_ETYPES, ET_DIM), jnp.float32),
            pltpu.VMEM((ECHUNK,), jnp.int32),
            pltpu.VMEM((ECHUNK,), jnp.int32),
            pltpu.VMEM((ETJ_OUT, ECHUNK_T, COL_T, ROW_T), jnp.float32),
            pltpu.VMEM((ETJ_OUT, ECHUNK_T, COL_T, ROW_T), jnp.float32),
            pltpu.SemaphoreType.DMA,
            pltpu.SemaphoreType.DMA,
            pltpu.SemaphoreType.DMA,
            pltpu.SemaphoreType.DMA,
            pltpu.SemaphoreType.DMA,
            pltpu.SemaphoreType.DMA,
            pltpu.SemaphoreType.DMA,
            pltpu.SemaphoreType.DMA,
            pltpu.SemaphoreType.DMA,
            pltpu.SemaphoreType.DMA,
        ],
    )
    return f(x, eattr4, ntypes, etypes, ntab, etab)


def kernel(x, eattr, ntypes, etypes, ntype_table, etype_table):
    ntypes = jnp.pad(ntypes.astype(jnp.int32),
                     (0, NPAD - N)).reshape(N_RT, ROW_T)
    etypes = etypes.astype(jnp.int32).reshape(N_ECHUNKS, ECHUNK)
    x_pad = jnp.pad(x, ((0, NPAD - N), (0, 0)))
    eattr4 = eattr.reshape(E_RT, ROW_T, ETJ_IN, COL_T).transpose(2, 0, 3, 1)
    xcat4, ecat4 = _run(x_pad, eattr4, ntypes, etypes,
                        ntype_table, etype_table)
    x_cat = xcat4.transpose(1, 3, 0, 2).reshape(NPAD, NT_DIM + D_FEAT)[:N]
    eattr_cat = ecat4.transpose(1, 3, 0, 2).reshape(E, ET_DIM + D_EDGE)
    return (x_cat, eattr_cat)
